# baseline (speedup 1.0000x reference)
import jax
import jax.numpy as jnp
from jax.experimental import pallas as pl

_B, _N = 4, 512
_IN_NF, _IN_EF, _OUT_F = 16, 8, 16
_PACK = 16
_RR = _N // _PACK
_TI = 64


def _fused_kernel(wp_ref, ap_ref, xp_ref, bd1_ref, b1_ref, bd2_ref, b2_ref,
                  bdn1_ref, bn1_ref, bdn2_ref, bn2_ref, e_ref, f_ref,
                  wout_ref, xout_ref):
    ti = _TI
    wp = wp_ref[0].reshape(ti * _RR, _PACK * _IN_EF)
    h = jnp.maximum(
        jnp.dot(wp, bd1_ref[...], preferred_element_type=jnp.float32)
        + b1_ref[...], 0.0)
    wnp = jnp.maximum(
        jnp.dot(h, bd2_ref[...], preferred_element_type=jnp.float32)
        + b2_ref[...], 0.0)
    wout_ref[0] = wnp.reshape(ti, _RR, _PACK * _OUT_F)

    xp = xp_ref[0]
    h1 = jnp.maximum(
        jnp.dot(xp, bdn1_ref[...], preferred_element_type=jnp.float32)
        + bn1_ref[...], 0.0)
    x1p = jnp.maximum(
        jnp.dot(h1, bdn2_ref[...], preferred_element_type=jnp.float32)
        + bn2_ref[...], 0.0)

    ap = ap_ref[0]
    s1 = jnp.sum(ap, axis=1)
    asum = jnp.sum(s1, axis=1, keepdims=True)
    inv = jnp.where(asum == 0.0, 0.0, 1.0 / asum)
    aexp = jnp.dot(ap.reshape(ti * _RR, _PACK), e_ref[...],
                   preferred_element_type=jnp.float32)
    t = wnp.reshape(ti, _RR, _PACK * _OUT_F) \
        * aexp.reshape(ti, _RR, _PACK * _OUT_F) * x1p[None]
    red = jnp.sum(t, axis=1)
    xnew = jnp.dot(red, f_ref[...],
                   preferred_element_type=jnp.float32) * inv
    xout_ref[0] = xnew


@jax.jit
def kernel(A, W, x, We1, be1, We2, be2, Wn1, bn1, Wn2, bn2):
    f32 = jnp.float32
    wp = W.reshape(_B, _N, _RR, _PACK * _IN_EF)
    ap = A.reshape(_B, _N, _RR, _PACK)
    xp = x.reshape(_B, _RR, _PACK * _IN_NF)

    eye = jnp.eye(_PACK, dtype=f32)
    bd1 = jnp.kron(eye, We1[:_IN_EF])
    bd2 = jnp.kron(eye, We2)
    bdn1 = jnp.kron(eye, Wn1)
    bdn2 = jnp.kron(eye, Wn2)
    b1 = jnp.tile(be1, _PACK)[None]
    b2 = jnp.tile(be2, _PACK)[None]
    bn1t = jnp.tile(bn1, _PACK)[None]
    bn2t = jnp.tile(bn2, _PACK)[None]
    sel_e = jnp.kron(eye, jnp.ones((1, _OUT_F), f32))
    sel_f = jnp.kron(jnp.ones((_PACK, 1), f32),
                     jnp.eye(_OUT_F, dtype=f32))

    const = lambda *shape: pl.BlockSpec(shape, lambda b, i: (0,) * len(shape))
    wout, xout = pl.pallas_call(
        _fused_kernel,
        grid=(_B, _N // _TI),
        in_specs=[
            pl.BlockSpec((1, _TI, _RR, _PACK * _IN_EF),
                         lambda b, i: (b, i, 0, 0)),
            pl.BlockSpec((1, _TI, _RR, _PACK), lambda b, i: (b, i, 0, 0)),
            pl.BlockSpec((1, _RR, _PACK * _IN_NF), lambda b, i: (b, 0, 0)),
            const(_PACK * _IN_EF, _PACK * _OUT_F),
            const(1, _PACK * _OUT_F),
            const(_PACK * _OUT_F, _PACK * _OUT_F),
            const(1, _PACK * _OUT_F),
            const(_PACK * _IN_NF, _PACK * _OUT_F),
            const(1, _PACK * _OUT_F),
            const(_PACK * _OUT_F, _PACK * _OUT_F),
            const(1, _PACK * _OUT_F),
            const(_PACK, _PACK * _OUT_F),
            const(_PACK * _OUT_F, _OUT_F),
        ],
        out_specs=[
            pl.BlockSpec((1, _TI, _RR, _PACK * _OUT_F),
                         lambda b, i: (b, i, 0, 0)),
            pl.BlockSpec((1, _TI, _OUT_F), lambda b, i: (b, i, 0)),
        ],
        out_shape=[
            jax.ShapeDtypeStruct((_B, _N, _RR, _PACK * _OUT_F), f32),
            jax.ShapeDtypeStruct((_B, _N, _OUT_F), f32),
        ],
    )(wp, ap, xp, bd1, b1, bd2, b2, bdn1, bn1t, bdn2, bn2t, sel_e, sel_f)
    return wout.reshape(_B, _N, _N, _OUT_F), xout
